# baseline (speedup 1.0000x reference)
import jax
import jax.numpy as jnp
from jax.experimental import pallas as pl
from jax.experimental.pallas import tpu as pltpu

N, DEG, D_IN, D_OUT = 10000, 32, 128, 128
BN = 1000


def _grn_block(neigh_ref, attn_ref, e_ref, w_ref, b_ref, out_ref):
    attn_exp = jax.lax.dot_general(
        attn_ref[...], e_ref[...],
        dimension_numbers=(((1,), (0,)), ((), ())),
        preferred_element_type=jnp.float32,
    )
    acc = neigh_ref[:, 0:D_IN] * attn_exp[:, 0:D_IN]
    for k in range(1, DEG):
        acc = acc + neigh_ref[:, k * D_IN:(k + 1) * D_IN] * attn_exp[:, k * D_IN:(k + 1) * D_IN]
    proj = jax.lax.dot_general(
        acc, w_ref[...],
        dimension_numbers=(((1,), (1,)), ((), ())),
        preferred_element_type=jnp.float32,
    )
    x = proj + b_ref[...][None, :]
    out_ref[...] = jnp.where(x > 0, x, jnp.exp(x) - 1.0)


def kernel(nodes, neighbors, attention_scores, W, b):
    del nodes
    neigh2d = neighbors.reshape(N, DEG * D_IN)
    col = jax.lax.broadcasted_iota(jnp.int32, (DEG, DEG * D_IN), 1)
    row = jax.lax.broadcasted_iota(jnp.int32, (DEG, DEG * D_IN), 0)
    expander = (col // D_IN == row).astype(jnp.float32)
    return pl.pallas_call(
        _grn_block,
        grid=(N // BN,),
        in_specs=[
            pl.BlockSpec((BN, DEG * D_IN), lambda i: (i, 0)),
            pl.BlockSpec((BN, DEG), lambda i: (i, 0)),
            pl.BlockSpec((DEG, DEG * D_IN), lambda i: (0, 0)),
            pl.BlockSpec((D_OUT, D_IN), lambda i: (0, 0)),
            pl.BlockSpec((D_OUT,), lambda i: (0,)),
        ],
        out_specs=pl.BlockSpec((BN, D_OUT), lambda i: (i, 0)),
        out_shape=jax.ShapeDtypeStruct((N, D_OUT), jnp.float32),
        compiler_params=pltpu.CompilerParams(
            dimension_semantics=("parallel",),
        ),
    )(neigh2d, attention_scores, expander, W, b)


# speedup vs baseline: 1.9076x; 1.9076x over previous
import functools

import jax
import jax.numpy as jnp
from jax import lax
from jax.experimental import pallas as pl
from jax.experimental.pallas import tpu as pltpu
from jax.experimental.pallas import tpu_sc as plsc

N, DEG, D_IN, D_OUT = 10000, 32, 128, 128

N_SC = 3328
N_TC = N - N_SC
BN = 1112

SC_CORES, SC_SUBCORES, SC_LANES = 2, 16, 16
NW = SC_CORES * SC_SUBCORES
PER_W = N_SC // NW
CHUNK = 8
D_CHUNKS = D_IN // SC_LANES


def _grn_block(neigh_ref, attn_ref, w_ref, b_ref, out_ref):
    neigh = neigh_ref[...]
    attn = attn_ref[...]
    agg = jnp.sum(neigh * attn[:, :, None], axis=1)
    proj = jax.lax.dot_general(
        agg, w_ref[...],
        dimension_numbers=(((1,), (1,)), ((), ())),
        preferred_element_type=jnp.float32,
    )
    x = proj + b_ref[...][None, :]
    out_ref[...] = jnp.where(x > 0, x, jnp.exp(x) - 1.0)


def _proj_block(agg_ref, w_ref, b_ref, out_ref):
    proj = jax.lax.dot_general(
        agg_ref[...], w_ref[...],
        dimension_numbers=(((1,), (1,)), ((), ())),
        preferred_element_type=jnp.float32,
    )
    x = proj + b_ref[...][None, :]
    out_ref[...] = jnp.where(x > 0, x, jnp.exp(x) - 1.0)


def _sc_agg_body(neigh_hbm, attn_hbm, agg_hbm, nbuf, abuf, obuf, sem_n, sem_a, sem_o):
    wid = lax.axis_index("s") * SC_CORES + lax.axis_index("c")
    base = N_TC + wid * PER_W

    @pl.loop(0, PER_W, step=CHUNK)
    def _chunk(i0):
        nb = base + i0
        pltpu.async_copy(neigh_hbm.at[pl.ds(nb, CHUNK)], nbuf, sem_n)
        pltpu.async_copy(attn_hbm.at[pl.ds(nb, CHUNK)], abuf, sem_a)
        pltpu.make_async_copy(neigh_hbm.at[pl.ds(nb, CHUNK)], nbuf, sem_n).wait()
        pltpu.make_async_copy(attn_hbm.at[pl.ds(nb, CHUNK)], abuf, sem_a).wait()

        @pl.loop(0, CHUNK)
        def _node(i):
            avecs = [abuf[i, pl.ds(j * SC_LANES, SC_LANES)]
                     for j in range(DEG // SC_LANES)]
            scalars = [avecs[k // SC_LANES][k % SC_LANES] for k in range(DEG)]
            for c in range(D_CHUNKS):
                acc = scalars[0] * nbuf[i, 0, pl.ds(c * SC_LANES, SC_LANES)]
                for k in range(1, DEG):
                    acc = acc + scalars[k] * nbuf[i, k, pl.ds(c * SC_LANES, SC_LANES)]
                obuf[i, pl.ds(c * SC_LANES, SC_LANES)] = acc

        pltpu.async_copy(obuf, agg_hbm.at[pl.ds(wid * PER_W + i0, CHUNK)], sem_o).wait()


def _sc_aggregate(neighbors, attention_scores):
    mesh = plsc.VectorSubcoreMesh(core_axis_name="c", subcore_axis_name="s")
    k = pl.kernel(
        _sc_agg_body,
        mesh=mesh,
        out_type=jax.ShapeDtypeStruct((N_SC, D_IN), jnp.float32),
        scratch_types=[
            pltpu.VMEM((CHUNK, DEG, D_IN), jnp.float32),
            pltpu.VMEM((CHUNK, DEG), jnp.float32),
            pltpu.VMEM((CHUNK, D_IN), jnp.float32),
            pltpu.SemaphoreType.DMA,
            pltpu.SemaphoreType.DMA,
            pltpu.SemaphoreType.DMA,
        ],
    )
    return k(neighbors, attention_scores)


def kernel(nodes, neighbors, attention_scores, W, b):
    del nodes

    agg_sc = _sc_aggregate(neighbors, attention_scores)

    out_head = pl.pallas_call(
        _grn_block,
        grid=(N_TC // BN,),
        in_specs=[
            pl.BlockSpec((BN, DEG, D_IN), lambda i: (i, 0, 0)),
            pl.BlockSpec((BN, DEG), lambda i: (i, 0)),
            pl.BlockSpec((D_OUT, D_IN), lambda i: (0, 0)),
            pl.BlockSpec((D_OUT,), lambda i: (0,)),
        ],
        out_specs=pl.BlockSpec((BN, D_OUT), lambda i: (i, 0)),
        out_shape=jax.ShapeDtypeStruct((N_TC, D_OUT), jnp.float32),
        compiler_params=pltpu.CompilerParams(
            dimension_semantics=("parallel",),
        ),
    )(neighbors, attention_scores, W, b)

    out_tail = pl.pallas_call(
        _proj_block,
        in_specs=[
            pl.BlockSpec((N_SC, D_IN), lambda: (0, 0)),
            pl.BlockSpec((D_OUT, D_IN), lambda: (0, 0)),
            pl.BlockSpec((D_OUT,), lambda: (0,)),
        ],
        out_specs=pl.BlockSpec((N_SC, D_OUT), lambda: (0, 0)),
        out_shape=jax.ShapeDtypeStruct((N_SC, D_OUT), jnp.float32),
    )(agg_sc, W, b)

    return jnp.concatenate([out_head, out_tail], axis=0)


# speedup vs baseline: 2.2603x; 1.1849x over previous
import jax
import jax.numpy as jnp
from jax import lax
from jax.experimental import pallas as pl
from jax.experimental.pallas import tpu as pltpu
from jax.experimental.pallas import tpu_sc as plsc

N, DEG, D_IN, D_OUT = 10000, 32, 128, 128

N_SC = 3328
N_TC = N - N_SC
BN = 1112

SC_CORES, SC_SUBCORES, SC_LANES = 2, 16, 16
NW = SC_CORES * SC_SUBCORES
PER_W = N_SC // NW
CHUNK = 8
NCHUNK = PER_W // CHUNK


def _grn_block(neigh_ref, attn_ref, w_ref, b_ref, out_ref):
    neigh = neigh_ref[...]
    attn = attn_ref[...]
    agg = jnp.sum(neigh * attn[:, :, None], axis=1)
    proj = jax.lax.dot_general(
        agg, w_ref[...],
        dimension_numbers=(((1,), (1,)), ((), ())),
        preferred_element_type=jnp.float32,
    )
    x = proj + b_ref[...][None, :]
    out_ref[...] = jnp.where(x > 0, x, jnp.exp(x) - 1.0)


def _proj_tail(full_ref, agg_ref, w_ref, b_ref, out_ref, scratch, sem):
    del full_ref
    proj = jax.lax.dot_general(
        agg_ref[...], w_ref[...],
        dimension_numbers=(((1,), (1,)), ((), ())),
        preferred_element_type=jnp.float32,
    )
    x = proj + b_ref[...][None, :]
    scratch[...] = jnp.where(x > 0, x, jnp.exp(x) - 1.0)
    pltpu.async_copy(scratch, out_ref.at[pl.ds(N_TC, N_SC)], sem).wait()


def _sc_compute_chunk(nbuf, abuf, obuf):
    @pl.loop(0, CHUNK)
    def _node(i):
        avecs = [abuf[i, pl.ds(j * SC_LANES, SC_LANES)]
                 for j in range(DEG // SC_LANES)]
        scalars = [avecs[k // SC_LANES][k % SC_LANES] for k in range(DEG)]
        for c in range(D_IN // SC_LANES):
            acc = scalars[0] * nbuf[i, 0, pl.ds(c * SC_LANES, SC_LANES)]
            for k in range(1, DEG):
                acc = acc + scalars[k] * nbuf[i, k, pl.ds(c * SC_LANES, SC_LANES)]
            obuf[i, pl.ds(c * SC_LANES, SC_LANES)] = acc


def _sc_agg_body(neigh_hbm, attn_hbm, agg_hbm,
                 nbuf0, nbuf1, abuf0, abuf1, obuf0, obuf1,
                 sem_n0, sem_n1, sem_a0, sem_a1, sem_o0, sem_o1):
    wid = lax.axis_index("s") * SC_CORES + lax.axis_index("c")
    base = wid * PER_W

    def issue_in(c_idx, nbuf, abuf, sem_n, sem_a):
        nb = base + c_idx * CHUNK
        pltpu.async_copy(neigh_hbm.at[pl.ds(N_TC + nb, CHUNK)], nbuf, sem_n)
        pltpu.async_copy(attn_hbm.at[pl.ds(nb, CHUNK)], abuf, sem_a)

    def wait_in(c_idx, nbuf, abuf, sem_n, sem_a):
        nb = base + c_idx * CHUNK
        pltpu.make_async_copy(neigh_hbm.at[pl.ds(N_TC + nb, CHUNK)], nbuf, sem_n).wait()
        pltpu.make_async_copy(attn_hbm.at[pl.ds(nb, CHUNK)], abuf, sem_a).wait()

    def flush_out(c_idx, obuf, sem_o):
        nb = base + c_idx * CHUNK
        pltpu.async_copy(obuf, agg_hbm.at[pl.ds(nb, CHUNK)], sem_o).wait()

    issue_in(0, nbuf0, abuf0, sem_n0, sem_a0)

    @pl.loop(0, NCHUNK // 2)
    def _pair(p):
        c0 = 2 * p
        wait_in(c0, nbuf0, abuf0, sem_n0, sem_a0)
        issue_in(c0 + 1, nbuf1, abuf1, sem_n1, sem_a1)
        _sc_compute_chunk(nbuf0, abuf0, obuf0)
        flush_out(c0, obuf0, sem_o0)
        wait_in(c0 + 1, nbuf1, abuf1, sem_n1, sem_a1)
        issue_in(c0 + 2, nbuf0, abuf0, sem_n0, sem_a0)
        _sc_compute_chunk(nbuf1, abuf1, obuf1)
        flush_out(c0 + 1, obuf1, sem_o1)

    last = NCHUNK - 1
    wait_in(last, nbuf0, abuf0, sem_n0, sem_a0)
    _sc_compute_chunk(nbuf0, abuf0, obuf0)
    flush_out(last, obuf0, sem_o0)


def _sc_aggregate(neighbors, attn_sc):
    mesh = plsc.VectorSubcoreMesh(core_axis_name="c", subcore_axis_name="s")
    vmem = pltpu.VMEM
    k = pl.kernel(
        _sc_agg_body,
        mesh=mesh,
        out_type=jax.ShapeDtypeStruct((N_SC, D_IN), jnp.float32),
        scratch_types=[
            vmem((CHUNK, DEG, D_IN), jnp.float32),
            vmem((CHUNK, DEG, D_IN), jnp.float32),
            vmem((CHUNK, DEG), jnp.float32),
            vmem((CHUNK, DEG), jnp.float32),
            vmem((CHUNK, D_IN), jnp.float32),
            vmem((CHUNK, D_IN), jnp.float32),
            pltpu.SemaphoreType.DMA,
            pltpu.SemaphoreType.DMA,
            pltpu.SemaphoreType.DMA,
            pltpu.SemaphoreType.DMA,
            pltpu.SemaphoreType.DMA,
            pltpu.SemaphoreType.DMA,
        ],
    )
    return k(neighbors, attn_sc)


def kernel(nodes, neighbors, attention_scores, W, b):
    del nodes

    attn_sc = lax.slice_in_dim(attention_scores, N_TC, N, axis=0)
    agg_sc = _sc_aggregate(neighbors, attn_sc)

    out_full = pl.pallas_call(
        _grn_block,
        grid=(N_TC // BN,),
        in_specs=[
            pl.BlockSpec((BN, DEG, D_IN), lambda i: (i, 0, 0)),
            pl.BlockSpec((BN, DEG), lambda i: (i, 0)),
            pl.BlockSpec((D_OUT, D_IN), lambda i: (0, 0)),
            pl.BlockSpec((D_OUT,), lambda i: (0,)),
        ],
        out_specs=pl.BlockSpec((BN, D_OUT), lambda i: (i, 0)),
        out_shape=jax.ShapeDtypeStruct((N, D_OUT), jnp.float32),
        compiler_params=pltpu.CompilerParams(
            dimension_semantics=("parallel",),
        ),
    )(neighbors, attention_scores, W, b)

    return pl.pallas_call(
        _proj_tail,
        in_specs=[
            pl.BlockSpec(memory_space=pl.ANY),
            pl.BlockSpec((N_SC, D_IN), lambda: (0, 0)),
            pl.BlockSpec((D_OUT, D_IN), lambda: (0, 0)),
            pl.BlockSpec((D_OUT,), lambda: (0,)),
        ],
        out_specs=pl.BlockSpec(memory_space=pl.ANY),
        out_shape=jax.ShapeDtypeStruct((N, D_OUT), jnp.float32),
        scratch_shapes=[
            pltpu.VMEM((N_SC, D_OUT), jnp.float32),
            pltpu.SemaphoreType.DMA,
        ],
        input_output_aliases={0: 0},
    )(out_full, agg_sc, W, b)


# speedup vs baseline: 2.9327x; 1.2975x over previous
import jax
import jax.numpy as jnp
from jax.experimental import pallas as pl
from jax.experimental.pallas import tpu as pltpu

N, DEG, D_IN, D_OUT = 10000, 32, 128, 128
BN = 1000
HD = DEG // 2


def _grn_block(neigh0_ref, neigh1_ref, attn_ref, w_ref, b_ref, out_ref):
    attn = attn_ref[...]
    agg = (
        jnp.sum(neigh0_ref[...] * attn[:, :HD, None], axis=1)
        + jnp.sum(neigh1_ref[...] * attn[:, HD:, None], axis=1)
    )
    proj = jax.lax.dot_general(
        agg, w_ref[...],
        dimension_numbers=(((1,), (1,)), ((), ())),
        preferred_element_type=jnp.float32,
    )
    x = proj + b_ref[...][None, :]
    out_ref[...] = jnp.where(x > 0, x, jnp.exp(x) - 1.0)


def kernel(nodes, neighbors, attention_scores, W, b):
    del nodes
    return pl.pallas_call(
        _grn_block,
        grid=(N // BN,),
        in_specs=[
            pl.BlockSpec((BN, HD, D_IN), lambda i: (i, 0, 0)),
            pl.BlockSpec((BN, HD, D_IN), lambda i: (i, 1, 0)),
            pl.BlockSpec((BN, DEG), lambda i: (i, 0)),
            pl.BlockSpec((D_OUT, D_IN), lambda i: (0, 0)),
            pl.BlockSpec((D_OUT,), lambda i: (0,)),
        ],
        out_specs=pl.BlockSpec((BN, D_OUT), lambda i: (i, 0)),
        out_shape=jax.ShapeDtypeStruct((N, D_OUT), jnp.float32),
        compiler_params=pltpu.CompilerParams(
            dimension_semantics=("parallel",),
        ),
    )(neighbors, neighbors, attention_scores, W, b)
